# baseline (speedup 1.0000x reference)
import jax
import jax.numpy as jnp
from jax.experimental import pallas as pl
from jax.experimental.pallas import tpu as pltpu


def _conv_kernel(ho, wo, x_ref, w_ref, b_ref, o_ref):
    c = x_ref.shape[1]
    xt = jnp.transpose(x_ref[0])
    xb = xt.astype(jnp.bfloat16)
    xi = pltpu.bitcast(xb, jnp.int32)
    planes = []
    for idx in range(2):
        p = pltpu.unpack_elementwise(
            xi, index=idx, packed_dtype=jnp.bfloat16, unpacked_dtype=jnp.float32)
        planes.append(p.astype(jnp.bfloat16).reshape(ho, 2, wo, c))

    zrow = jnp.zeros((1, wo, c), jnp.bfloat16)
    zcol = jnp.zeros((ho, 1, c), jnp.bfloat16)
    phase = [[planes[pw][:, ph] for pw in range(2)] for ph in range(2)]
    jshift = [jnp.concatenate([phase[ph][0][:, 1:], zcol], axis=1)
              for ph in range(2)]

    taps = []
    for kh in range(3):
        for kw in range(3):
            t = jshift[kh % 2] if kw == 2 else phase[kh % 2][kw]
            if kh == 2:
                t = jnp.concatenate([t[1:], zrow], axis=0)
            taps.append(t.reshape(ho * wo, c))
    patches = jnp.concatenate(taps, axis=1)
    acc = jnp.dot(patches, w_ref[...], preferred_element_type=jnp.float32)
    o_ref[0] = jnp.transpose(acc + b_ref[...])


def kernel(x, w, b):
    B, C, H, W = x.shape
    Co = w.shape[0]
    Ho, Wo = H // 2, W // 2
    N = Ho * Wo

    xu = x.reshape(B, C, H * W)

    w_mat = jnp.transpose(w, (2, 3, 1, 0)).reshape(9 * C, Co).astype(jnp.bfloat16)
    b_row = b.reshape(1, Co)

    out = pl.pallas_call(
        lambda *refs: _conv_kernel(Ho, Wo, *refs),
        out_shape=jax.ShapeDtypeStruct((B, Co, N), jnp.float32),
        grid=(2, B // 2),
        in_specs=[
            pl.BlockSpec((1, C, H * W), lambda ci, i: (ci * (B // 2) + i, 0, 0)),
            pl.BlockSpec((9 * C, Co), lambda ci, i: (0, 0)),
            pl.BlockSpec((1, Co), lambda ci, i: (0, 0)),
        ],
        out_specs=pl.BlockSpec((1, Co, N), lambda ci, i: (ci * (B // 2) + i, 0, 0)),
        compiler_params=pltpu.CompilerParams(
            dimension_semantics=("parallel", "arbitrary"),
            vmem_limit_bytes=64 * 1024 * 1024,
        ),
    )(xu, w_mat, b_row)

    return out.reshape(B, Co, Ho, Wo)


# speedup vs baseline: 1.1673x; 1.1673x over previous
import jax
import jax.numpy as jnp
from jax.experimental import pallas as pl
from jax.experimental.pallas import tpu as pltpu


def _conv_kernel(ho, wo, x_ref, w_ref, b_ref, o_ref):
    c = x_ref.shape[1]
    xt = jnp.transpose(x_ref[0])
    xb = xt.astype(jnp.bfloat16)
    xi = pltpu.bitcast(xb, jnp.int32)
    planes = []
    for idx in range(2):
        p = pltpu.unpack_elementwise(
            xi, index=idx, packed_dtype=jnp.bfloat16, unpacked_dtype=jnp.float32)
        planes.append(p.astype(jnp.bfloat16).reshape(ho, 2, wo, c))

    zrow = jnp.zeros((1, wo, c), jnp.bfloat16)
    zcol = jnp.zeros((ho, 1, c), jnp.bfloat16)
    phase = [[planes[pw][:, ph] for pw in range(2)] for ph in range(2)]
    jshift = [jnp.concatenate([phase[ph][0][:, 1:], zcol], axis=1)
              for ph in range(2)]

    taps = []
    for kh in range(3):
        for kw in range(3):
            t = jshift[kh % 2] if kw == 2 else phase[kh % 2][kw]
            if kh == 2:
                t = jnp.concatenate([t[1:], zrow], axis=0)
            taps.append(t.reshape(ho * wo, c))
    patches = jnp.concatenate(taps, axis=1)
    acc = jnp.dot(patches, w_ref[...], preferred_element_type=jnp.float32)
    o_ref[0] = acc + b_ref[...]


def kernel(x, w, b):
    B, C, H, W = x.shape
    Co = w.shape[0]
    Ho, Wo = H // 2, W // 2
    N = Ho * Wo

    xu = x.reshape(B, C, H * W)

    w_mat = jnp.transpose(w, (2, 3, 1, 0)).reshape(9 * C, Co).astype(jnp.bfloat16)
    b_row = b.reshape(1, Co)

    out = pl.pallas_call(
        lambda *refs: _conv_kernel(Ho, Wo, *refs),
        out_shape=jax.ShapeDtypeStruct((B, N, Co), jnp.float32),
        grid=(2, B // 2),
        in_specs=[
            pl.BlockSpec((1, C, H * W), lambda ci, i: (ci * (B // 2) + i, 0, 0)),
            pl.BlockSpec((9 * C, Co), lambda ci, i: (0, 0)),
            pl.BlockSpec((1, Co), lambda ci, i: (0, 0)),
        ],
        out_specs=pl.BlockSpec((1, N, Co), lambda ci, i: (ci * (B // 2) + i, 0, 0)),
        compiler_params=pltpu.CompilerParams(
            dimension_semantics=("parallel", "arbitrary"),
            vmem_limit_bytes=64 * 1024 * 1024,
        ),
    )(xu, w_mat, b_row)

    return out.transpose(0, 2, 1).reshape(B, Co, Ho, Wo)
